# baseline (speedup 1.0000x reference)
import functools

import jax
import jax.numpy as jnp
from jax import lax
from jax.experimental import pallas as pl
from jax.experimental.pallas import tpu as pltpu
from jax.experimental.pallas import tpu_sc as plsc

N_ENT = 1000000
N_REL = 500
DIM = 64
D2 = 2 * DIM
B = 4096
K = 32

NC, NS = 2, 16
NW = NC * NS
CH = 128
NBUF = 4
ROWS_H = B * K
PER_W = ROWS_H // NW
N_ITER = PER_W // (CH * NBUF)
PER_W_B = B // NW
L = 16

BBLK = 256
GRID = B // BBLK
NPAIR = BBLK * K // 2
RPAD = 512


def _sc_gather_body(h_hbm, r_hbm, idx_t, idx_a, idx_r, outh, outa,
                    idxbuf, rb0, rb1, rb2, rb3, pk0, pk1, pk2, pk3,
                    gsem, ssem):
    rowbufs = (rb0, rb1, rb2, rb3)
    pkbufs = (pk0, pk1, pk2, pk3)
    wid = lax.axis_index("s") * NC + lax.axis_index("c")
    base = wid * PER_W

    def loop_body(j, carry):
        off = base + j * (CH * NBUF)
        pltpu.sync_copy(idx_t.at[pl.ds(off, CH * NBUF)], idxbuf)
        handles = [
            pltpu.async_copy(h_hbm.at[idxbuf.at[pl.ds(b * CH, CH)]],
                             rowbufs[b], gsem)
            for b in range(NBUF)
        ]
        for h in handles:
            h.wait()

        for b in range(NBUF):
            rb, pk = rowbufs[b], pkbufs[b]

            def pack_body(p, c, rb=rb, pk=pk):
                for v in range(DIM // L):
                    pk[p, pl.ds(v * L, L)] = rb[2 * p, pl.ds(v * L, L)]
                    pk[p, pl.ds(DIM + v * L, L)] = rb[2 * p + 1, pl.ds(v * L, L)]
                return c

            lax.fori_loop(0, CH // 2, pack_body, 0)

        stores = [
            pltpu.async_copy(pkbufs[b],
                             outh.at[pl.ds((off + b * CH) // 2, CH // 2)],
                             ssem)
            for b in range(NBUF)
        ]
        for s in stores:
            s.wait()
        return carry

    lax.fori_loop(0, N_ITER, loop_body, 0)

    boff = wid * PER_W_B
    pltpu.sync_copy(idx_a.at[pl.ds(boff, PER_W_B)], idxbuf.at[pl.ds(0, PER_W_B)])
    pltpu.sync_copy(idx_r.at[pl.ds(boff, PER_W_B)], idxbuf.at[pl.ds(CH, PER_W_B)])
    ca = pltpu.async_copy(h_hbm.at[idxbuf.at[pl.ds(0, PER_W_B)]], rowbufs[0], gsem)
    cr = pltpu.async_copy(r_hbm.at[idxbuf.at[pl.ds(CH, PER_W_B)]], rowbufs[1], gsem)
    ca.wait()
    cr.wait()

    def pack_ar(p, c):
        for v in range(DIM // L):
            pk0[p, pl.ds(v * L, L)] = rowbufs[0][p, pl.ds(v * L, L)]
            pk0[p, pl.ds(DIM + v * L, L)] = rowbufs[1][p, pl.ds(v * L, L)]
        return c

    lax.fori_loop(0, CH // 2, pack_ar, 0)

    def pack_ar2(p, c):
        for v in range(DIM // L):
            pk1[p, pl.ds(v * L, L)] = rowbufs[0][CH // 2 + p, pl.ds(v * L, L)]
            pk1[p, pl.ds(DIM + v * L, L)] = rowbufs[1][CH // 2 + p, pl.ds(v * L, L)]
        return c

    lax.fori_loop(0, CH // 2, pack_ar2, 0)
    pltpu.sync_copy(pk0, outa.at[pl.ds(boff, CH // 2)])
    pltpu.sync_copy(pk1, outa.at[pl.ds(boff + CH // 2, CH // 2)])


@functools.lru_cache(maxsize=None)
def _build_sc_gather():
    return pl.kernel(
        _sc_gather_body,
        out_type=[
            jax.ShapeDtypeStruct((ROWS_H // 2, D2), jnp.float32),
            jax.ShapeDtypeStruct((B, D2), jnp.float32),
        ],
        mesh=plsc.VectorSubcoreMesh(core_axis_name="c", subcore_axis_name="s",
                                    num_cores=NC, num_subcores=NS),
        scratch_types=(
            [pltpu.VMEM((CH * NBUF,), jnp.int32)]
            + [pltpu.VMEM((CH, DIM), jnp.float32) for _ in range(NBUF)]
            + [pltpu.VMEM((CH // 2, D2), jnp.float32) for _ in range(NBUF)]
            + [pltpu.SemaphoreType.DMA, pltpu.SemaphoreType.DMA]
        ),
        compiler_params=pltpu.CompilerParams(use_tc_tiling_on_sc=False),
    )


def _tc_body(ht_ref, ar_ref, pre_ref, pro_ref, rhi_ref, rlo_ref,
             w1a_ref, bd1_ref, bd2_ref, b1_ref, b2_ref, out_ref):
    f32 = jnp.float32
    pre = pre_ref[0, 0, :]
    pro = pro_ref[0, 0, :]
    iota = lax.broadcasted_iota(jnp.int32, (NPAIR, RPAD), 1)
    oh = jnp.concatenate(
        [(iota == pre.reshape(NPAIR, 1)).astype(jnp.bfloat16),
         (iota == pro.reshape(NPAIR, 1)).astype(jnp.bfloat16)], axis=1)
    rt = jnp.dot(oh, rhi_ref[...], preferred_element_type=f32)
    rt = rt + jnp.dot(oh, rlo_ref[...], preferred_element_type=f32)

    ar = ar_ref[...]
    a = ar[:, :DIM]
    rq = ar[:, DIM:]
    u0 = lax.dot_general(rq, w1a_ref[...], (((1,), (1,)), ((), ())),
                         preferred_element_type=f32,
                         precision=lax.Precision.HIGHEST)
    u0d = jnp.concatenate([u0, u0], axis=1)
    u0e = jnp.broadcast_to(u0d.reshape(BBLK, 1, D2),
                           (BBLK, K // 2, D2)).reshape(NPAIR, D2)
    vk = jnp.dot(rt, bd1_ref[...], preferred_element_type=f32,
                 precision=lax.Precision.HIGHEST)
    act = jnp.maximum(u0e + vk + b1_ref[...], 0.0)
    m = jnp.dot(act, bd2_ref[...], preferred_element_type=f32,
                precision=lax.Precision.HIGHEST) + b2_ref[...]

    ht = ht_ref[...]
    p1 = jnp.sum((m * ht).reshape(BBLK, K // 2, D2), axis=1)
    p2 = jnp.sum((m * rt).reshape(BBLK, K // 2, D2), axis=1)
    s1 = p1[:, :DIM] + p1[:, DIM:]
    s2 = p2[:, :DIM] + p2[:, DIM:]
    fre = s1 - a * s2
    query = a * rq
    refn = jnp.sum(jnp.abs(fre), axis=1, keepdims=True)
    qn = jnp.sum(jnp.abs(query), axis=1, keepdims=True)
    out_ref[0] = query + fre / (1e-9 + refn / qn * 2.5)


def _tc_call(ht, ar, pre, pro, rhi, rlo, w1a, bd1, bd2, b1, b2):
    return pl.pallas_call(
        _tc_body,
        grid=(GRID,),
        in_specs=[
            pl.BlockSpec((NPAIR, D2), lambda i: (i, 0)),
            pl.BlockSpec((BBLK, D2), lambda i: (i, 0)),
            pl.BlockSpec((1, 1, NPAIR), lambda i: (i, 0, 0)),
            pl.BlockSpec((1, 1, NPAIR), lambda i: (i, 0, 0)),
            pl.BlockSpec((2 * RPAD, D2), lambda i: (0, 0)),
            pl.BlockSpec((2 * RPAD, D2), lambda i: (0, 0)),
            pl.BlockSpec((DIM, DIM), lambda i: (0, 0)),
            pl.BlockSpec((D2, D2), lambda i: (0, 0)),
            pl.BlockSpec((D2, D2), lambda i: (0, 0)),
            pl.BlockSpec((1, D2), lambda i: (0, 0)),
            pl.BlockSpec((1, D2), lambda i: (0, 0)),
        ],
        out_specs=pl.BlockSpec((1, BBLK, DIM), lambda i: (i, 0, 0)),
        out_shape=jax.ShapeDtypeStruct((GRID, BBLK, DIM), jnp.float32),
    )(ht, ar, pre, pro, rhi, rlo, w1a, bd1, bd2, b1, b2)


def kernel(h_table, r_table, W1, b1, W2, b2, anchors, rel_0, p1_target, p1_rel):
    f32 = jnp.float32
    idx_t = p1_target.reshape(-1).astype(jnp.int32)
    idx_a = anchors.astype(jnp.int32)
    idx_r = rel_0.astype(jnp.int32)
    ht2, ar_rows = _build_sc_gather()(h_table, r_table, idx_t, idx_a, idx_r)

    rhi = r_table.astype(jnp.bfloat16)
    rlo = (r_table - rhi.astype(f32)).astype(jnp.bfloat16)
    z = jnp.zeros((RPAD - (N_REL + 1), DIM), jnp.bfloat16)
    zc = jnp.zeros((RPAD, DIM), jnp.bfloat16)
    rhi_cat = jnp.concatenate(
        [jnp.concatenate([jnp.concatenate([rhi, z], 0), zc], 1),
         jnp.concatenate([zc, jnp.concatenate([rhi, z], 0)], 1)], axis=0)
    rlo_cat = jnp.concatenate(
        [jnp.concatenate([jnp.concatenate([rlo, z], 0), zc], 1),
         jnp.concatenate([zc, jnp.concatenate([rlo, z], 0)], 1)], axis=0)

    w1a = W1[:, :DIM]
    w1bt = W1[:, DIM:].T
    zw = jnp.zeros((DIM, DIM), f32)
    bd1 = jnp.concatenate(
        [jnp.concatenate([w1bt, zw], 1), jnp.concatenate([zw, w1bt], 1)], 0)
    w2t = W2.T
    bd2 = jnp.concatenate(
        [jnp.concatenate([w2t, zw], 1), jnp.concatenate([zw, w2t], 1)], 0)
    b1c = jnp.concatenate([b1, b1]).reshape(1, D2)
    b2c = jnp.concatenate([b2, b2]).reshape(1, D2)

    pre = p1_rel[:, 0::2].astype(jnp.int32).reshape(GRID, 1, NPAIR)
    pro = p1_rel[:, 1::2].astype(jnp.int32).reshape(GRID, 1, NPAIR)

    out = _tc_call(
        ht2, ar_rows,
        pre, pro, rhi_cat, rlo_cat, w1a, bd1, bd2, b1c, b2c,
    )
    return out.reshape(B, DIM)


# speedup vs baseline: 1.0013x; 1.0013x over previous
import functools

import jax
import jax.numpy as jnp
from jax import lax
from jax.experimental import pallas as pl
from jax.experimental.pallas import tpu as pltpu
from jax.experimental.pallas import tpu_sc as plsc

N_ENT = 1000000
N_REL = 500
DIM = 64
D2 = 2 * DIM
B = 4096
K = 32

NC, NS = 2, 16
NW = NC * NS
CH = 128
NBUF = 4
ROWS_H = B * K
PER_W = ROWS_H // NW
N_ITER = PER_W // (CH * NBUF)
PER_W_B = B // NW
L = 16

BBLK = 256
GRID = B // BBLK
NPAIR = BBLK * K // 2
RPAD = 512


def _sc_gather_body(h_hbm, r_hbm, idx_t, idx_a, idx_r, outh, outa,
                    idxbuf, rb0, rb1, rb2, rb3, pk0, pk1, pk2, pk3,
                    gsem, ssem):
    rowbufs = (rb0, rb1, rb2, rb3)
    pkbufs = (pk0, pk1, pk2, pk3)
    wid = lax.axis_index("s") * NC + lax.axis_index("c")
    base = wid * PER_W

    def loop_body(j, carry):
        off = base + j * (CH * NBUF)
        pltpu.sync_copy(idx_t.at[pl.ds(off, CH * NBUF)], idxbuf)
        handles = [
            pltpu.async_copy(h_hbm.at[idxbuf.at[pl.ds(b * CH, CH)]],
                             rowbufs[b], gsem)
            for b in range(NBUF)
        ]
        for h in handles:
            h.wait()

        for b in range(NBUF):
            rb, pk = rowbufs[b], pkbufs[b]

            def pack_body(p, c, rb=rb, pk=pk):
                for v in range(DIM // L):
                    pk[pl.ds(p * D2 + v * L, L)] = rb[2 * p, pl.ds(v * L, L)]
                    pk[pl.ds(p * D2 + DIM + v * L, L)] = rb[2 * p + 1, pl.ds(v * L, L)]
                return c

            lax.fori_loop(0, CH // 2, pack_body, 0)

        stores = [
            pltpu.async_copy(pkbufs[b],
                             outh.at[pl.ds((off + b * CH) * DIM, CH * DIM)],
                             ssem)
            for b in range(NBUF)
        ]
        for s in stores:
            s.wait()
        return carry

    lax.fori_loop(0, N_ITER, loop_body, 0)

    boff = wid * PER_W_B
    pltpu.sync_copy(idx_a.at[pl.ds(boff, PER_W_B)], idxbuf.at[pl.ds(0, PER_W_B)])
    pltpu.sync_copy(idx_r.at[pl.ds(boff, PER_W_B)], idxbuf.at[pl.ds(CH, PER_W_B)])
    ca = pltpu.async_copy(h_hbm.at[idxbuf.at[pl.ds(0, PER_W_B)]], rowbufs[0], gsem)
    cr = pltpu.async_copy(r_hbm.at[idxbuf.at[pl.ds(CH, PER_W_B)]], rowbufs[1], gsem)
    ca.wait()
    cr.wait()

    def pack_ar(p, c):
        for v in range(DIM // L):
            pk0[pl.ds(p * D2 + v * L, L)] = rowbufs[0][p, pl.ds(v * L, L)]
            pk0[pl.ds(p * D2 + DIM + v * L, L)] = rowbufs[1][p, pl.ds(v * L, L)]
        return c

    lax.fori_loop(0, CH // 2, pack_ar, 0)

    def pack_ar2(p, c):
        for v in range(DIM // L):
            pk1[pl.ds(p * D2 + v * L, L)] = rowbufs[0][CH // 2 + p, pl.ds(v * L, L)]
            pk1[pl.ds(p * D2 + DIM + v * L, L)] = rowbufs[1][CH // 2 + p, pl.ds(v * L, L)]
        return c

    lax.fori_loop(0, CH // 2, pack_ar2, 0)
    pltpu.sync_copy(pk0, outa.at[pl.ds(boff * D2, CH // 2 * D2)])
    pltpu.sync_copy(pk1, outa.at[pl.ds(boff * D2 + CH // 2 * D2, CH // 2 * D2)])


@functools.lru_cache(maxsize=None)
def _build_sc_gather():
    return pl.kernel(
        _sc_gather_body,
        out_type=[
            jax.ShapeDtypeStruct((ROWS_H * DIM,), jnp.float32),
            jax.ShapeDtypeStruct((B * D2,), jnp.float32),
        ],
        mesh=plsc.VectorSubcoreMesh(core_axis_name="c", subcore_axis_name="s",
                                    num_cores=NC, num_subcores=NS),
        scratch_types=(
            [pltpu.VMEM((CH * NBUF,), jnp.int32)]
            + [pltpu.VMEM((CH, DIM), jnp.float32) for _ in range(NBUF)]
            + [pltpu.VMEM((CH // 2 * D2,), jnp.float32) for _ in range(NBUF)]
            + [pltpu.SemaphoreType.DMA, pltpu.SemaphoreType.DMA]
        ),
        compiler_params=pltpu.CompilerParams(use_tc_tiling_on_sc=False),
    )


def _tc_body(ht_ref, ar_ref, pre_ref, pro_ref, rhi_ref, rlo_ref,
             w1a_ref, bd1_ref, bd2_ref, b1_ref, b2_ref, out_ref):
    f32 = jnp.float32
    pre = pre_ref[0, 0, :]
    pro = pro_ref[0, 0, :]
    iota = lax.broadcasted_iota(jnp.int32, (NPAIR, RPAD), 1)
    oh = jnp.concatenate(
        [(iota == pre.reshape(NPAIR, 1)).astype(jnp.bfloat16),
         (iota == pro.reshape(NPAIR, 1)).astype(jnp.bfloat16)], axis=1)
    rt = jnp.dot(oh, rhi_ref[...], preferred_element_type=f32)
    rt = rt + jnp.dot(oh, rlo_ref[...], preferred_element_type=f32)

    ar = ar_ref[...].reshape(BBLK, D2)
    a = ar[:, :DIM]
    rq = ar[:, DIM:]
    u0 = lax.dot_general(rq, w1a_ref[...], (((1,), (1,)), ((), ())),
                         preferred_element_type=f32,
                         precision=lax.Precision.HIGHEST)
    u0d = jnp.concatenate([u0, u0], axis=1)
    u0e = jnp.broadcast_to(u0d.reshape(BBLK, 1, D2),
                           (BBLK, K // 2, D2)).reshape(NPAIR, D2)
    vk = jnp.dot(rt, bd1_ref[...], preferred_element_type=f32,
                 precision=lax.Precision.HIGHEST)
    act = jnp.maximum(u0e + vk + b1_ref[...], 0.0)
    m = jnp.dot(act, bd2_ref[...], preferred_element_type=f32,
                precision=lax.Precision.HIGHEST) + b2_ref[...]

    ht = ht_ref[...].reshape(NPAIR, D2)
    p1 = jnp.sum((m * ht).reshape(BBLK, K // 2, D2), axis=1)
    p2 = jnp.sum((m * rt).reshape(BBLK, K // 2, D2), axis=1)
    s1 = p1[:, :DIM] + p1[:, DIM:]
    s2 = p2[:, :DIM] + p2[:, DIM:]
    fre = s1 - a * s2
    query = a * rq
    refn = jnp.sum(jnp.abs(fre), axis=1, keepdims=True)
    qn = jnp.sum(jnp.abs(query), axis=1, keepdims=True)
    out_ref[0] = query + fre / (1e-9 + refn / qn * 2.5)


def _tc_call(ht, ar, pre, pro, rhi, rlo, w1a, bd1, bd2, b1, b2):
    return pl.pallas_call(
        _tc_body,
        grid=(GRID,),
        in_specs=[
            pl.BlockSpec((NPAIR * D2,), lambda i: (i,)),
            pl.BlockSpec((BBLK * D2,), lambda i: (i,)),
            pl.BlockSpec((1, 1, NPAIR), lambda i: (i, 0, 0)),
            pl.BlockSpec((1, 1, NPAIR), lambda i: (i, 0, 0)),
            pl.BlockSpec((2 * RPAD, D2), lambda i: (0, 0)),
            pl.BlockSpec((2 * RPAD, D2), lambda i: (0, 0)),
            pl.BlockSpec((DIM, DIM), lambda i: (0, 0)),
            pl.BlockSpec((D2, D2), lambda i: (0, 0)),
            pl.BlockSpec((D2, D2), lambda i: (0, 0)),
            pl.BlockSpec((1, D2), lambda i: (0, 0)),
            pl.BlockSpec((1, D2), lambda i: (0, 0)),
        ],
        out_specs=pl.BlockSpec((1, BBLK, DIM), lambda i: (i, 0, 0)),
        out_shape=jax.ShapeDtypeStruct((GRID, BBLK, DIM), jnp.float32),
    )(ht, ar, pre, pro, rhi, rlo, w1a, bd1, bd2, b1, b2)


def kernel(h_table, r_table, W1, b1, W2, b2, anchors, rel_0, p1_target, p1_rel):
    f32 = jnp.float32
    idx_t = p1_target.reshape(-1).astype(jnp.int32)
    idx_a = anchors.astype(jnp.int32)
    idx_r = rel_0.astype(jnp.int32)
    ht2, ar_rows = _build_sc_gather()(h_table, r_table, idx_t, idx_a, idx_r)

    rhi = r_table.astype(jnp.bfloat16)
    rlo = (r_table - rhi.astype(f32)).astype(jnp.bfloat16)
    z = jnp.zeros((RPAD - (N_REL + 1), DIM), jnp.bfloat16)
    zc = jnp.zeros((RPAD, DIM), jnp.bfloat16)
    rhi_cat = jnp.concatenate(
        [jnp.concatenate([jnp.concatenate([rhi, z], 0), zc], 1),
         jnp.concatenate([zc, jnp.concatenate([rhi, z], 0)], 1)], axis=0)
    rlo_cat = jnp.concatenate(
        [jnp.concatenate([jnp.concatenate([rlo, z], 0), zc], 1),
         jnp.concatenate([zc, jnp.concatenate([rlo, z], 0)], 1)], axis=0)

    w1a = W1[:, :DIM]
    w1bt = W1[:, DIM:].T
    zw = jnp.zeros((DIM, DIM), f32)
    bd1 = jnp.concatenate(
        [jnp.concatenate([w1bt, zw], 1), jnp.concatenate([zw, w1bt], 1)], 0)
    w2t = W2.T
    bd2 = jnp.concatenate(
        [jnp.concatenate([w2t, zw], 1), jnp.concatenate([zw, w2t], 1)], 0)
    b1c = jnp.concatenate([b1, b1]).reshape(1, D2)
    b2c = jnp.concatenate([b2, b2]).reshape(1, D2)

    pre = p1_rel[:, 0::2].astype(jnp.int32).reshape(GRID, 1, NPAIR)
    pro = p1_rel[:, 1::2].astype(jnp.int32).reshape(GRID, 1, NPAIR)

    out = _tc_call(
        ht2, ar_rows,
        pre, pro, rhi_cat, rlo_cat, w1a, bd1, bd2, b1c, b2c,
    )
    return out.reshape(B, DIM)
